# Initial kernel scaffold; baseline (speedup 1.0000x reference)
import jax
import jax.numpy as jnp
from jax.experimental import pallas as pl


def kernel(x, edge_index, edge_attr, W, b):
    raise NotImplementedError("write your pallas kernel here")


# speedup vs baseline: 5.0147x; 5.0147x over previous
import functools

import jax
import jax.numpy as jnp
from jax import lax
from jax.experimental import pallas as pl
from jax.experimental.pallas import tpu as pltpu
from jax.experimental.pallas import tpu_sc as plsc

_EPS = 1e-05

_N_NODES = 10000
_N_EDGES = 320000
_D = 128
_NW = 32
_CHUNK = 80
_CHUNKS_PER_TILE = 128
_EDGES_PER_TILE = _CHUNK * _CHUNKS_PER_TILE
_N_EDGES_PAD = _EDGES_PER_TILE * _NW
_REAL_CHUNKS = _N_EDGES // _CHUNK
_IDX_BLK = 32
_ZROWS = 16
_ROWS_PER_SUB = 624
_ROWS_REMAINDER = _N_NODES - 16 * _ROWS_PER_SUB


def _sc_edge_stage(x, src2d, dst2d, edge_attr):
    mesh = plsc.VectorSubcoreMesh(core_axis_name="c", subcore_axis_name="s")

    @functools.partial(
        pl.kernel,
        mesh=mesh,
        out_type=[
            jax.ShapeDtypeStruct((_N_NODES, _D), jnp.float32),
            jax.ShapeDtypeStruct((_N_NODES, _D), jnp.float32),
        ],
        scratch_types=[
            pltpu.VMEM((_IDX_BLK, _CHUNK), jnp.int32),
            pltpu.VMEM((_IDX_BLK, _CHUNK), jnp.int32),
            pltpu.VMEM((_CHUNK, _D), jnp.float32),
            pltpu.VMEM((_CHUNK, _D), jnp.float32),
            pltpu.VMEM_SHARED((_N_NODES, _D), jnp.float32),
            pltpu.SemaphoreType.DMA,
            pltpu.SemaphoreType.DMA,
        ],
    )
    def k(x_hbm, src_hbm, dst_hbm, ea_hbm, out0, out1,
          src_v, dst_v, rows_v, ea_v, aggr_sh, sem0, sem1):
        c = lax.axis_index("c")
        s = lax.axis_index("s")
        wid = c * 16 + s

        zv = jnp.zeros((16,), jnp.float32)

        def zrow(r, carry):
            for kk in range(_D // 16):
                rows_v[r, pl.ds(kk * 16, 16)] = zv
            return carry

        lax.fori_loop(0, _ZROWS, zrow, 0)
        row_base = s * _ROWS_PER_SUB
        n_zchunks = jnp.where(s == 15, (_ROWS_PER_SUB + _ROWS_REMAINDER) // _ZROWS,
                              _ROWS_PER_SUB // _ZROWS)
        zsrc = rows_v.at[pl.ds(0, _ZROWS)]

        def zcopy(i, carry):
            pltpu.sync_copy(zsrc, aggr_sh.at[pl.ds(row_base + i * _ZROWS, _ZROWS)])
            return carry

        lax.fori_loop(0, n_zchunks, zcopy, 0)
        plsc.subcore_barrier()

        edge_base = wid * _EDGES_PER_TILE
        n_chunks = jnp.minimum(_CHUNKS_PER_TILE,
                               _REAL_CHUNKS - wid * _CHUNKS_PER_TILE)
        n_blks = n_chunks // _IDX_BLK

        def blk_body(bi, carry):
            pltpu.sync_copy(src_hbm.at[wid, pl.ds(bi * _IDX_BLK, _IDX_BLK)], src_v)
            pltpu.sync_copy(dst_hbm.at[wid, pl.ds(bi * _IDX_BLK, _IDX_BLK)], dst_v)

            def chunk_body(jj, ccarry):
                j = bi * _IDX_BLK + jj
                gat = pltpu.async_copy(x_hbm.at[src_v.at[jj]], rows_v, sem0)
                ea = pltpu.async_copy(
                    ea_hbm.at[pl.ds(edge_base + j * _CHUNK, _CHUNK)], ea_v, sem1)
                gat.wait()
                ea.wait()

                def row_body(r, rcarry):
                    for kk in range(_D // 16):
                        sl = pl.ds(kk * 16, 16)
                        rows_v[r, sl] = jnp.maximum(rows_v[r, sl] + ea_v[r, sl], 0.0)
                    return rcarry

                lax.fori_loop(0, _CHUNK, row_body, 0)
                pltpu.sync_copy(rows_v, aggr_sh.at[dst_v.at[jj]], add=True)
                return ccarry

            lax.fori_loop(0, _IDX_BLK, chunk_body, 0)
            return carry

        lax.fori_loop(0, n_blks, blk_body, 0)
        plsc.subcore_barrier()

        osl = pl.ds(row_base, _ROWS_PER_SUB)
        tail = pl.ds(16 * _ROWS_PER_SUB, _ROWS_REMAINDER)

        @pl.when(c == 0)
        def _():
            pltpu.sync_copy(aggr_sh.at[osl], out0.at[osl])

            @pl.when(s == 15)
            def _():
                pltpu.sync_copy(aggr_sh.at[tail], out0.at[tail])

        @pl.when(c == 1)
        def _():
            pltpu.sync_copy(aggr_sh.at[osl], out1.at[osl])

            @pl.when(s == 15)
            def _():
                pltpu.sync_copy(aggr_sh.at[tail], out1.at[tail])

    return k(x, src2d, dst2d, edge_attr)


def _tc_epilogue(x, p0, p1, W, b):
    blk = 2000

    def body(x_ref, p0_ref, p1_ref, w_ref, b_ref, o_ref):
        h = (1.0 + _EPS) * x_ref[...] + p0_ref[...] + p1_ref[...]
        o = lax.dot_general(h, w_ref[...], (((1,), (1,)), ((), ())),
                            preferred_element_type=jnp.float32)
        o_ref[...] = jnp.maximum(o + b_ref[...], 0.0) + x_ref[...]

    return pl.pallas_call(
        body,
        grid=(_N_NODES // blk,),
        in_specs=[
            pl.BlockSpec((blk, _D), lambda i: (i, 0)),
            pl.BlockSpec((blk, _D), lambda i: (i, 0)),
            pl.BlockSpec((blk, _D), lambda i: (i, 0)),
            pl.BlockSpec((_D, _D), lambda i: (0, 0)),
            pl.BlockSpec((1, _D), lambda i: (0, 0)),
        ],
        out_specs=pl.BlockSpec((blk, _D), lambda i: (i, 0)),
        out_shape=jax.ShapeDtypeStruct((_N_NODES, _D), jnp.float32),
    )(x, p0, p1, W, b.reshape(1, _D))


def kernel(x, edge_index, edge_attr, W, b):
    pad = _N_EDGES_PAD - _N_EDGES
    src3d = jnp.pad(edge_index[0].astype(jnp.int32), (0, pad)).reshape(
        _NW, _CHUNKS_PER_TILE, _CHUNK)
    dst3d = jnp.pad(edge_index[1].astype(jnp.int32), (0, pad)).reshape(
        _NW, _CHUNKS_PER_TILE, _CHUNK)
    p0, p1 = _sc_edge_stage(x, src3d, dst3d, edge_attr)
    return _tc_epilogue(x, p0, p1, W, b)
